# baseline (speedup 1.0000x reference)
import functools

import numpy as np

import jax
import jax.numpy as jnp
from jax import lax
from jax.experimental import pallas as pl
from jax.experimental.pallas import tpu as pltpu
from jax.experimental.pallas import tpu_sc as plsc

BATCH = 4096
SEQ = 200
VOCAB = 100000
D_EMBED = 128
D_OUT = 64

NUM_CORES = 2
NUM_SUBCORES = 16
NUM_WORKERS = NUM_CORES * NUM_SUBCORES
ROWS_PER_W = BATCH // NUM_WORKERS
CHUNK_A = 128
CHUNK_B = SEQ - CHUNK_A
LANES = 16
PACKED_WORDS = D_OUT // 2

BV = 2000

_g = np.concatenate([
    np.arange(0, 32, 2), np.arange(1, 32, 2),
    np.arange(32, 64, 2), np.arange(33, 64, 2)])
_INV_G = np.argsort(_g)


def _proj_body(tab_ref, wt_ref, out_ref):
    i = pl.program_id(0)
    x = tab_ref[...]
    row_ids = lax.broadcasted_iota(jnp.int32, x.shape, 0) + i * BV
    x = jnp.where(row_ids == 0, jnp.float32(0.0), x)
    out_ref[...] = lax.dot_general(
        x, wt_ref[...], (((1,), (0,)), ((), ())),
        preferred_element_type=jnp.float32).astype(jnp.bfloat16)


def _project(table, wt):
    return pl.pallas_call(
        _proj_body,
        grid=(VOCAB // BV,),
        in_specs=[
            pl.BlockSpec((BV, D_EMBED), lambda i: (i, 0)),
            pl.BlockSpec((D_EMBED, D_OUT), lambda i: (0, 0)),
        ],
        out_specs=pl.BlockSpec((BV, D_OUT), lambda i: (i, 0)),
        out_shape=jax.ShapeDtypeStruct((VOCAB, D_OUT), jnp.bfloat16),
    )(table, wt)


def _bag_body(p_hbm, idx_hbm, b_hbm, out_hbm,
              idx_v, rows_v, out_v, b_v, sem0, sem1):
    wid = lax.axis_index("s") * NUM_CORES + lax.axis_index("c")
    base = wid * ROWS_PER_W
    pltpu.sync_copy(idx_hbm.at[pl.ds(base, ROWS_PER_W)], idx_v)
    pltpu.sync_copy(b_hbm, b_v)
    bias = [b_v[pl.ds(LANES * k, LANES)] for k in range(4)]
    inv_seq = jnp.float32(1.0 / SEQ)
    sems = (sem0, sem1)

    def issue(r, buf):
        pltpu.async_copy(
            p_hbm.at[idx_v.at[r, pl.ds(0, CHUNK_A)]],
            rows_v.at[buf, pl.ds(0, CHUNK_A), :],
            sems[buf])
        pltpu.async_copy(
            p_hbm.at[idx_v.at[r, pl.ds(CHUNK_A, CHUNK_B)]],
            rows_v.at[buf, pl.ds(CHUNK_A, CHUNK_B), :],
            sems[buf])

    def drain(buf):
        pltpu.make_async_copy(
            p_hbm.at[pl.ds(0, SEQ)], rows_v.at[buf], sems[buf]).wait()

    def reduce_into(r_out, buf):
        def body(j, acc):
            a0, b0 = plsc.unpack(
                plsc.bitcast(rows_v[buf, j, pl.ds(0, LANES)], jnp.bfloat16),
                format=plsc.PackFormat.INTERLEAVED)
            a1, b1 = plsc.unpack(
                plsc.bitcast(rows_v[buf, j, pl.ds(LANES, LANES)], jnp.bfloat16),
                format=plsc.PackFormat.INTERLEAVED)
            return (acc[0] + a0, acc[1] + b0, acc[2] + a1, acc[3] + b1)
        zero = jnp.zeros((LANES,), jnp.float32)
        acc = lax.fori_loop(0, SEQ, body, (zero,) * 4, unroll=8)
        for k in range(4):
            out_v[r_out, pl.ds(LANES * k, LANES)] = acc[k] * inv_seq + bias[k]

    issue(0, 0)

    @pl.loop(0, ROWS_PER_W, step=2)
    def _(r):
        issue(r + 1, 1)
        drain(0)
        reduce_into(r, 0)

        @pl.when(r + 2 < ROWS_PER_W)
        def _():
            issue(r + 2, 0)

        drain(1)
        reduce_into(r + 1, 1)

    pltpu.sync_copy(out_v, out_hbm.at[pl.ds(base, ROWS_PER_W)])


@functools.partial(
    pl.kernel,
    out_type=jax.ShapeDtypeStruct((BATCH, D_OUT), jnp.float32),
    mesh=plsc.VectorSubcoreMesh(core_axis_name="c", subcore_axis_name="s"),
    compiler_params=pltpu.CompilerParams(
        use_tc_tiling_on_sc=False, needs_layout_passes=False),
    scratch_types=[
        pltpu.VMEM((ROWS_PER_W, SEQ), jnp.int32),
        pltpu.VMEM((2, SEQ, PACKED_WORDS), jnp.int32),
        pltpu.VMEM((ROWS_PER_W, D_OUT), jnp.float32),
        pltpu.VMEM((D_OUT,), jnp.float32),
        pltpu.SemaphoreType.DMA,
        pltpu.SemaphoreType.DMA,
    ],
)
def _bag(p_hbm, idx_hbm, b_hbm, out_hbm,
         idx_v, rows_v, out_v, b_v, sem0, sem1):
    _bag_body(p_hbm, idx_hbm, b_hbm, out_hbm,
              idx_v, rows_v, out_v, b_v, sem0, sem1)


@jax.jit
def kernel(idx_words, embed_table, W, b):
    wt = W.T[:, _INV_G]
    proj = _project(embed_table, wt)
    packed = lax.bitcast_convert_type(
        proj.reshape(VOCAB, PACKED_WORDS, 2), jnp.int32)
    idx = idx_words.astype(jnp.int32)
    return _bag(packed, idx, b)


# speedup vs baseline: 2.0011x; 2.0011x over previous
import functools

import numpy as np

import jax
import jax.numpy as jnp
from jax import lax
from jax.experimental import pallas as pl
from jax.experimental.pallas import tpu as pltpu
from jax.experimental.pallas import tpu_sc as plsc

BATCH = 4096
SEQ = 200
VOCAB = 100000
D_EMBED = 128
D_OUT = 64

NUM_CORES = 2
NUM_SUBCORES = 16
NUM_WORKERS = NUM_CORES * NUM_SUBCORES
ROWS_PER_W = BATCH // NUM_WORKERS
CHUNK_A = 128
CHUNK_B = SEQ - CHUNK_A
LANES = 16

BV = 2000

_g = np.concatenate([
    np.arange(0, 32, 2), np.arange(1, 32, 2),
    np.arange(32, 64, 2), np.arange(33, 64, 2)])
_INV_G = np.argsort(_g)


def _proj_body(tab_ref, wt_ref, out_ref):
    i = pl.program_id(0)
    x = tab_ref[...]
    row_ids = lax.broadcasted_iota(jnp.int32, x.shape, 0) + i * BV
    x = jnp.where(row_ids == 0, jnp.float32(0.0), x)
    out_ref[...] = lax.dot_general(
        x, wt_ref[...], (((1,), (0,)), ((), ())),
        preferred_element_type=jnp.float32).astype(jnp.bfloat16)


def _project(table, wt):
    return pl.pallas_call(
        _proj_body,
        grid=(VOCAB // BV,),
        in_specs=[
            pl.BlockSpec((BV, D_EMBED), lambda i: (i, 0)),
            pl.BlockSpec((D_EMBED, D_OUT), lambda i: (0, 0)),
        ],
        out_specs=pl.BlockSpec((BV, D_OUT), lambda i: (i, 0)),
        out_shape=jax.ShapeDtypeStruct((VOCAB, D_OUT), jnp.bfloat16),
    )(table, wt)


def _bag_body(p_hbm, idx_hbm, b_hbm, out_hbm,
              idx_v, rows_v, out_v, b_v, sem0, sem1):
    wid = lax.axis_index("s") * NUM_CORES + lax.axis_index("c")
    base = wid * ROWS_PER_W
    pltpu.sync_copy(idx_hbm.at[pl.ds(base, ROWS_PER_W)], idx_v)
    pltpu.sync_copy(b_hbm, b_v)
    bias = [b_v[pl.ds(LANES * k, LANES)] for k in range(4)]
    inv_seq = jnp.float32(1.0 / SEQ)
    sems = (sem0, sem1)

    def issue(r, buf):
        pltpu.async_copy(
            p_hbm.at[idx_v.at[r, pl.ds(0, CHUNK_A)]],
            rows_v.at[buf, pl.ds(0, CHUNK_A), :],
            sems[buf])
        pltpu.async_copy(
            p_hbm.at[idx_v.at[r, pl.ds(CHUNK_A, CHUNK_B)]],
            rows_v.at[buf, pl.ds(CHUNK_A, CHUNK_B), :],
            sems[buf])

    def drain(buf):
        pltpu.make_async_copy(
            p_hbm.at[pl.ds(0, SEQ)], rows_v.at[buf], sems[buf]).wait()

    def reduce_into(r_out, buf):
        def body(j, acc):
            a0, b0 = plsc.unpack(
                rows_v[buf, j, pl.ds(0, 2 * LANES)],
                format=plsc.PackFormat.INTERLEAVED)
            a1, b1 = plsc.unpack(
                rows_v[buf, j, pl.ds(2 * LANES, 2 * LANES)],
                format=plsc.PackFormat.INTERLEAVED)
            return (acc[0] + a0, acc[1] + b0, acc[2] + a1, acc[3] + b1)
        zero = jnp.zeros((LANES,), jnp.float32)
        acc = lax.fori_loop(0, SEQ, body, (zero,) * 4, unroll=8)
        for k in range(4):
            out_v[r_out, pl.ds(LANES * k, LANES)] = acc[k] * inv_seq + bias[k]

    issue(0, 0)

    @pl.loop(0, ROWS_PER_W, step=2)
    def _(r):
        issue(r + 1, 1)
        drain(0)
        reduce_into(r, 0)

        @pl.when(r + 2 < ROWS_PER_W)
        def _():
            issue(r + 2, 0)

        drain(1)
        reduce_into(r + 1, 1)

    pltpu.sync_copy(out_v, out_hbm.at[pl.ds(base, ROWS_PER_W)])


@functools.partial(
    pl.kernel,
    out_type=jax.ShapeDtypeStruct((BATCH, D_OUT), jnp.float32),
    mesh=plsc.VectorSubcoreMesh(core_axis_name="c", subcore_axis_name="s"),
    compiler_params=pltpu.CompilerParams(
        use_tc_tiling_on_sc=False, needs_layout_passes=False),
    scratch_types=[
        pltpu.VMEM((ROWS_PER_W, SEQ), jnp.int32),
        pltpu.VMEM((2, SEQ, D_OUT), jnp.bfloat16),
        pltpu.VMEM((ROWS_PER_W, D_OUT), jnp.float32),
        pltpu.VMEM((D_OUT,), jnp.float32),
        pltpu.SemaphoreType.DMA,
        pltpu.SemaphoreType.DMA,
    ],
)
def _bag(p_hbm, idx_hbm, b_hbm, out_hbm,
         idx_v, rows_v, out_v, b_v, sem0, sem1):
    _bag_body(p_hbm, idx_hbm, b_hbm, out_hbm,
              idx_v, rows_v, out_v, b_v, sem0, sem1)


@jax.jit
def kernel(idx_words, embed_table, W, b):
    wt = W.T[:, _INV_G]
    proj = _project(embed_table, wt)
    idx = idx_words.astype(jnp.int32)
    return _bag(proj, idx, b)


# speedup vs baseline: 2.0861x; 1.0425x over previous
import functools

import numpy as np

import jax
import jax.numpy as jnp
from jax import lax
from jax.experimental import pallas as pl
from jax.experimental.pallas import tpu as pltpu
from jax.experimental.pallas import tpu_sc as plsc

BATCH = 4096
SEQ = 200
VOCAB = 100000
D_EMBED = 128
D_OUT = 64

NUM_CORES = 2
NUM_SUBCORES = 16
NUM_WORKERS = NUM_CORES * NUM_SUBCORES
ROWS_PER_W = BATCH // NUM_WORKERS
CHUNK_A = 128
CHUNK_B = SEQ - CHUNK_A
LANES = 16
PACKED_WORDS = D_OUT // 2

BV = 4000
BV4 = BV // 4

_E_COLS = np.concatenate([np.arange(0, 16), np.arange(32, 48)])
_O_COLS = np.concatenate([np.arange(16, 32), np.arange(48, 64)])


def _pack_bf16_pair(e_f32, o_f32):
    e_bits = lax.bitcast_convert_type(
        e_f32.astype(jnp.bfloat16).astype(jnp.float32), jnp.uint32)
    o_bits = lax.bitcast_convert_type(
        o_f32.astype(jnp.bfloat16).astype(jnp.float32), jnp.uint32)
    word = lax.shift_right_logical(e_bits, jnp.uint32(16)) | (
        o_bits & jnp.uint32(0xFFFF0000))
    return lax.bitcast_convert_type(word, jnp.int32)


def _proj_body(tab_ref, we_ref, wo_ref, out_ref):
    i = pl.program_id(0)
    x = tab_ref[...]
    row_ids = lax.broadcasted_iota(jnp.int32, x.shape, 0) + i * BV4
    col_ids = lax.broadcasted_iota(jnp.int32, x.shape, 1)
    x = jnp.where((row_ids == 0) & (col_ids < D_EMBED), jnp.float32(0.0), x)
    parts = []
    for k in range(4):
        xk = x[:, k * D_EMBED:(k + 1) * D_EMBED]
        e = lax.dot_general(xk, we_ref[...], (((1,), (0,)), ((), ())),
                            preferred_element_type=jnp.float32)
        o = lax.dot_general(xk, wo_ref[...], (((1,), (0,)), ((), ())),
                            preferred_element_type=jnp.float32)
        parts.append(_pack_bf16_pair(e, o))
    out_ref[...] = jnp.concatenate(parts, axis=1)


def _project(table4, we, wo):
    return pl.pallas_call(
        _proj_body,
        grid=(VOCAB // BV,),
        in_specs=[
            pl.BlockSpec((BV4, 4 * D_EMBED), lambda i: (i, 0)),
            pl.BlockSpec((D_EMBED, PACKED_WORDS), lambda i: (0, 0)),
            pl.BlockSpec((D_EMBED, PACKED_WORDS), lambda i: (0, 0)),
        ],
        out_specs=pl.BlockSpec((BV4, 4 * PACKED_WORDS), lambda i: (i, 0)),
        out_shape=jax.ShapeDtypeStruct((VOCAB // 4, 4 * PACKED_WORDS),
                                       jnp.int32),
    )(table4, we, wo)


def _bag_body(p_hbm, idx_hbm, b_hbm, out_hbm,
              idx_v, rows_v, out_v, b_v, sem0, sem1):
    wid = lax.axis_index("s") * NUM_CORES + lax.axis_index("c")
    base = wid * ROWS_PER_W
    pltpu.sync_copy(idx_hbm.at[pl.ds(base, ROWS_PER_W)], idx_v)
    pltpu.sync_copy(b_hbm, b_v)
    bias = [b_v[pl.ds(LANES * k, LANES)] for k in range(4)]
    inv_seq = jnp.float32(1.0 / SEQ)
    sems = (sem0, sem1)

    def issue(r, buf):
        pltpu.async_copy(
            p_hbm.at[idx_v.at[r, pl.ds(0, CHUNK_A)]],
            rows_v.at[buf, pl.ds(0, CHUNK_A), :],
            sems[buf])
        pltpu.async_copy(
            p_hbm.at[idx_v.at[r, pl.ds(CHUNK_A, CHUNK_B)]],
            rows_v.at[buf, pl.ds(CHUNK_A, CHUNK_B), :],
            sems[buf])

    def drain(buf):
        pltpu.make_async_copy(
            p_hbm.at[pl.ds(0, SEQ)], rows_v.at[buf], sems[buf]).wait()

    def reduce_into(r_out, buf):
        def body(j, acc):
            a0, b0 = plsc.unpack(
                plsc.bitcast(rows_v[buf, j, pl.ds(0, LANES)], jnp.bfloat16),
                format=plsc.PackFormat.INTERLEAVED)
            a1, b1 = plsc.unpack(
                plsc.bitcast(rows_v[buf, j, pl.ds(LANES, LANES)],
                             jnp.bfloat16),
                format=plsc.PackFormat.INTERLEAVED)
            return (acc[0] + a0, acc[1] + b0, acc[2] + a1, acc[3] + b1)
        zero = jnp.zeros((LANES,), jnp.float32)
        acc = lax.fori_loop(0, SEQ, body, (zero,) * 4, unroll=8)
        for k in range(4):
            out_v[r_out, pl.ds(LANES * k, LANES)] = acc[k] * inv_seq + bias[k]

    issue(0, 0)

    @pl.loop(0, ROWS_PER_W, step=2)
    def _(r):
        issue(r + 1, 1)
        drain(0)
        reduce_into(r, 0)

        @pl.when(r + 2 < ROWS_PER_W)
        def _():
            issue(r + 2, 0)

        drain(1)
        reduce_into(r + 1, 1)

    pltpu.sync_copy(out_v, out_hbm.at[pl.ds(base, ROWS_PER_W)])


@functools.partial(
    pl.kernel,
    out_type=jax.ShapeDtypeStruct((BATCH, D_OUT), jnp.float32),
    mesh=plsc.VectorSubcoreMesh(core_axis_name="c", subcore_axis_name="s"),
    compiler_params=pltpu.CompilerParams(
        use_tc_tiling_on_sc=False, needs_layout_passes=False),
    scratch_types=[
        pltpu.VMEM((ROWS_PER_W, SEQ), jnp.int32),
        pltpu.VMEM((2, SEQ, PACKED_WORDS), jnp.int32),
        pltpu.VMEM((ROWS_PER_W, D_OUT), jnp.float32),
        pltpu.VMEM((D_OUT,), jnp.float32),
        pltpu.SemaphoreType.DMA,
        pltpu.SemaphoreType.DMA,
    ],
)
def _bag(p_hbm, idx_hbm, b_hbm, out_hbm,
         idx_v, rows_v, out_v, b_v, sem0, sem1):
    _bag_body(p_hbm, idx_hbm, b_hbm, out_hbm,
              idx_v, rows_v, out_v, b_v, sem0, sem1)


@jax.jit
def kernel(idx_words, embed_table, W, b):
    wt = W.T
    we = wt[:, _E_COLS]
    wo = wt[:, _O_COLS]
    table4 = embed_table.reshape(VOCAB // 4, 4 * D_EMBED)
    packed4 = _project(table4, we, wo)
    packed = packed4.reshape(VOCAB, PACKED_WORDS)
    idx = idx_words.astype(jnp.int32)
    return _bag(packed, idx, b)
